# baseline (speedup 1.0000x reference)
import functools

import jax
import jax.numpy as jnp
import numpy as np
from jax import lax
from jax.experimental import pallas as pl
from jax.experimental.pallas import tpu as pltpu
from jax.experimental.pallas import tpu_sc as plsc

BS = 4096
N_DIST = 20
N_FEAT = 26
DIM = 32
NH = 128

ROWS = BS * N_DIST
PAIRS = ROWS // 2
NCHUNK = 13
TOTAL_IDX = ROWS * N_FEAT

NW = 32
IDX_PER_W = TOTAL_IDX // NW
GL = 128
K = 13
NCH = IDX_PER_W // (K * GL)
KROWS = NCH * K


def _dest_rows():
    r = np.arange(TOTAL_IDX, dtype=np.int64)
    p, rem = r // 52, r % 52
    c, j4 = rem // 4, rem % 4
    dest = c * (4 * PAIRS) + p * 4 + j4
    return jnp.asarray(dest.astype(np.int32).reshape(NW, KROWS, GL))


VROWS = 1000000
VPAD = 1000064
TR_C = 512
TR_MAIN = 1952
TR_PER_W = TR_MAIN // NW


def _sc_transpose_table(tableT, tailT):
    mesh = plsc.VectorSubcoreMesh(core_axis_name="c", subcore_axis_name="s")

    @functools.partial(
        pl.kernel,
        out_type=jax.ShapeDtypeStruct((VPAD * DIM,), jnp.float32),
        mesh=mesh,
        compiler_params=pltpu.CompilerParams(needs_layout_passes=False),
        scratch_types=[
            pltpu.VMEM((2, DIM, TR_C), jnp.float32),
            pltpu.VMEM((2, TR_C * DIM), jnp.float32),
            pltpu.VMEM((DIM, 128), jnp.float32),
            pltpu.SemaphoreType.DMA,
            pltpu.SemaphoreType.DMA,
            pltpu.SemaphoreType.DMA,
            pltpu.SemaphoreType.DMA,
        ],
    )
    def body(tt_hbm, tail_hbm, out_hbm, in_v, out_v, tail_v,
             isem0, isem1, osem0, osem1):
        wid = lax.axis_index("s") * 2 + lax.axis_index("c")
        isems = (isem0, isem1)
        osems = (osem0, osem1)
        e_lo = lax.broadcasted_iota(jnp.int32, (16,), 0)
        e_hi = e_lo + 16

        def col0(ch):
            return (ch * NW + wid) * TR_C

        def in_copy(ch, b):
            return (tt_hbm.at[:, pl.ds(col0(ch), TR_C)], in_v.at[b], isems[b])

        def out_copy(ch, b):
            return (out_v.at[b],
                    out_hbm.at[pl.ds(col0(ch) * DIM, TR_C * DIM)],
                    osems[b])

        def transpose_rows(b, nrows):
            @pl.loop(0, nrows, unroll=8)
            def _row(r):
                rr = lax.broadcast(r, (16,))
                lo = plsc.load_gather(in_v.at[b], [e_lo, rr])
                hi = plsc.load_gather(in_v.at[b], [e_hi, rr])
                out_v[b, pl.ds(r * DIM, 16)] = lo
                out_v[b, pl.ds(r * DIM + 16, 16)] = hi

        pltpu.async_copy(*in_copy(0, 0))

        @pl.loop(0, (TR_PER_W - 1) // 2)
        def _step(i):
            for b in (0, 1):
                ch = 2 * i + b
                nb = 1 - b
                pltpu.async_copy(*in_copy(ch + 1, nb))
                pltpu.make_async_copy(*in_copy(ch, b)).wait()

                @pl.when(ch >= 2)
                def _reclaim():
                    pltpu.make_async_copy(*out_copy(ch - 2, b)).wait()

                transpose_rows(b, TR_C)
                pltpu.async_copy(*out_copy(ch, b))

        pltpu.make_async_copy(*in_copy(TR_PER_W - 1, 0)).wait()
        pltpu.make_async_copy(*out_copy(TR_PER_W - 3, 0)).wait()
        transpose_rows(0, TR_C)
        pltpu.async_copy(*out_copy(TR_PER_W - 1, 0))

        pltpu.make_async_copy(*out_copy(TR_PER_W - 2, 1)).wait()
        pltpu.make_async_copy(*out_copy(TR_PER_W - 1, 0)).wait()

        @pl.when(wid == 0)
        def _tail0():
            pltpu.sync_copy(tt_hbm.at[:, pl.ds(TR_MAIN * TR_C, TR_C)],
                            in_v.at[0])
            transpose_rows(0, TR_C)
            pltpu.sync_copy(out_v.at[0],
                            out_hbm.at[pl.ds(TR_MAIN * TR_C * DIM,
                                             TR_C * DIM)])

        @pl.when(wid == 1)
        def _tail1():
            base = TR_MAIN * TR_C + TR_C
            pltpu.sync_copy(tail_hbm, tail_v)

            @pl.loop(0, 128)
            def _row(r):
                rr = lax.broadcast(r, (16,))
                lo = plsc.load_gather(tail_v, [e_lo, rr])
                hi = plsc.load_gather(tail_v, [e_hi, rr])
                out_v[0, pl.ds(r * DIM, 16)] = lo
                out_v[0, pl.ds(r * DIM + 16, 16)] = hi

            pltpu.sync_copy(out_v.at[0, pl.ds(0, 128 * DIM)],
                            out_hbm.at[pl.ds(base * DIM, 128 * DIM)])

    return body(tableT, tailT)


def _sc_gather(idx3, table):
    mesh = plsc.VectorSubcoreMesh(core_axis_name="c", subcore_axis_name="s")

    @functools.partial(
        pl.kernel,
        out_type=jax.ShapeDtypeStruct((TOTAL_IDX, DIM), jnp.float32),
        mesh=mesh,
        compiler_params=pltpu.CompilerParams(use_tc_tiling_on_sc=False),
        scratch_types=[
            pltpu.VMEM((2, K, GL), jnp.int32),
            pltpu.VMEM((2, K, GL), jnp.int32),
            pltpu.VMEM((2, K * GL, DIM), jnp.float32),
            pltpu.SemaphoreType.DMA,
            pltpu.SemaphoreType.DMA,
            pltpu.SemaphoreType.DMA,
            pltpu.SemaphoreType.DMA,
        ],
    )
    def body(idx_hbm, oidx_hbm, table_hbm, out_hbm,
             idx_v, oidx_v, rows_v, gsem0, gsem1, ssem0, ssem1):
        wid = lax.axis_index("s") * 2 + lax.axis_index("c")
        gsems = (gsem0, gsem1)
        ssems = (ssem0, ssem1)

        def gather_copies(b, j):
            return (table_hbm.at[idx_v.at[b, j]],
                    rows_v.at[b, pl.ds(j * GL, GL)], gsems[b])

        def scatter_copies(b, j):
            return (rows_v.at[b, pl.ds(j * GL, GL)],
                    out_hbm.at[oidx_v.at[b, j]], ssems[b])

        def fire_gather(ch, b):
            pltpu.sync_copy(idx_hbm.at[wid, pl.ds(ch * K, K)], idx_v.at[b])
            pltpu.sync_copy(oidx_hbm.at[wid, pl.ds(ch * K, K)], oidx_v.at[b])
            for j in range(K):
                pltpu.async_copy(*gather_copies(b, j))

        def drain_gather(b):
            for j in range(K):
                pltpu.make_async_copy(*gather_copies(b, j)).wait()

        def fire_scatter(b):
            for j in range(K):
                pltpu.async_copy(*scatter_copies(b, j))

        def drain_scatter(b):
            for j in range(K):
                pltpu.make_async_copy(*scatter_copies(b, j)).wait()

        fire_gather(0, 0)

        @pl.loop(0, NCH // 2)
        def _step(i):
            for b in (0, 1):
                ch = 2 * i + b
                nb = 1 - b
                drain_gather(b)
                fire_scatter(b)

                @pl.when(ch + 1 < NCH)
                def _prefetch():
                    @pl.when(ch >= 1)
                    def _reclaim():
                        drain_scatter(nb)
                    fire_gather(ch + 1, nb)

        drain_scatter(0)
        drain_scatter(1)

    return body(idx3, _dest_rows(), table)


def _tc_score(g, x, idx, weo, b2):
    PB = 640
    BB = PB // 10

    def body(g_ref, x_ref, idx_ref, w_ref, b_ref, oe_ref, oo_ref):
        gb = g_ref[...].astype(jnp.bfloat16)
        wb = w_ref[...]
        acc = jnp.zeros((PB, 2 * NH), jnp.float32)
        for c in range(NCHUNK):
            acc += jnp.dot(gb[c], wb[c], preferred_element_type=jnp.float32)
        h = jnp.tanh(acc + b_ref[...])
        xb3 = jnp.broadcast_to(x_ref[...][:, None, :], (BB, 10, NH))
        he3 = h[:, :NH].reshape(BB, 10, NH)
        ho3 = h[:, NH:].reshape(BB, 10, NH)
        de = jnp.sum(he3 * xb3, axis=-1)
        do = jnp.sum(ho3 * xb3, axis=-1)
        az = jnp.all(idx_ref[...] == 0, axis=-1).astype(jnp.float32)
        dsel = lax.broadcasted_iota(jnp.int32, (N_DIST, 10), 0)
        ksel = lax.broadcasted_iota(jnp.int32, (N_DIST, 10), 1)
        se = (dsel == 2 * ksel).astype(jnp.float32)
        so = (dsel == 2 * ksel + 1).astype(jnp.float32)
        me = jnp.dot(az, se, preferred_element_type=jnp.float32) > 0.5
        mo = jnp.dot(az, so, preferred_element_type=jnp.float32) > 0.5
        oe_ref[...] = jnp.where(me, -jnp.inf, de)
        oo_ref[...] = jnp.where(mo, -jnp.inf, do)

    return pl.pallas_call(
        body,
        grid=(PAIRS // PB,),
        in_specs=[
            pl.BlockSpec((NCHUNK, PB, NH), lambda i: (0, i, 0)),
            pl.BlockSpec((BB, NH), lambda i: (i, 0)),
            pl.BlockSpec((BB, N_DIST, N_FEAT), lambda i: (i, 0, 0)),
            pl.BlockSpec((NCHUNK, NH, 2 * NH), lambda i: (0, 0, 0)),
            pl.BlockSpec((1, 2 * NH), lambda i: (0, 0)),
        ],
        out_specs=[
            pl.BlockSpec((BB, 10), lambda i: (i, 0)),
            pl.BlockSpec((BB, 10), lambda i: (i, 0)),
        ],
        out_shape=[
            jax.ShapeDtypeStruct((BS, 10), jnp.float32),
            jax.ShapeDtypeStruct((BS, 10), jnp.float32),
        ],
    )(g, x, idx, weo, b2)


def kernel(x, _input, table, W, b):
    idx3 = _input.reshape(NW, KROWS, GL)
    tailT = jnp.pad(lax.slice(table, (999936, 0), (VROWS, DIM)),
                    ((0, 64), (0, 0))).T
    tflat = _sc_transpose_table(table.T, tailT)
    g = _sc_gather(idx3, tflat.reshape(VPAD, DIM)).reshape(NCHUNK, PAIRS, NH)

    zeros = jnp.zeros_like(W)
    weo = jnp.concatenate(
        [jnp.concatenate([W, zeros], axis=0).reshape(NCHUNK, NH, NH),
         jnp.concatenate([zeros, W], axis=0).reshape(NCHUNK, NH, NH)],
        axis=-1,
    ).astype(jnp.bfloat16)
    b2 = jnp.concatenate([b, b]).reshape(1, 2 * NH)

    oe, oo = _tc_score(g, x, _input, weo, b2)
    return jnp.stack([oe, oo], axis=-1).reshape(BS, N_DIST)


# speedup vs baseline: 1.2798x; 1.2798x over previous
import functools

import jax
import jax.numpy as jnp
import numpy as np
from jax import lax
from jax.experimental import pallas as pl
from jax.experimental.pallas import tpu as pltpu
from jax.experimental.pallas import tpu_sc as plsc

BS = 4096
N_DIST = 20
N_FEAT = 26
DIM = 32
NH = 128

NHALF = 2
BS_H = BS // NHALF
ROWS_H = BS_H * N_DIST
PAIRS_H = ROWS_H // 2
NCHUNK = 13
IDX_H = ROWS_H * N_FEAT

NW = 32
IDX_PER_W = IDX_H // NW
GL = 128
K = 13
NCH = IDX_PER_W // (K * GL)
KROWS = NCH * K


def _dest_rows():
    r = np.arange(IDX_H, dtype=np.int64)
    p, rem = r // 52, r % 52
    c, j4 = rem // 4, rem % 4
    dest = c * (4 * PAIRS_H) + p * 4 + j4
    return jnp.asarray(dest.astype(np.int32).reshape(NW, KROWS, GL))


def _sc_gather(idx3, oidx3, table):
    mesh = plsc.VectorSubcoreMesh(core_axis_name="c", subcore_axis_name="s")

    @functools.partial(
        pl.kernel,
        out_type=jax.ShapeDtypeStruct((IDX_H, DIM), jnp.float32),
        mesh=mesh,
        compiler_params=pltpu.CompilerParams(use_tc_tiling_on_sc=False),
        scratch_types=[
            pltpu.VMEM((2, K, GL), jnp.int32),
            pltpu.VMEM((2, K, GL), jnp.int32),
            pltpu.VMEM((2, K * GL, DIM), jnp.float32),
            pltpu.SemaphoreType.DMA,
            pltpu.SemaphoreType.DMA,
            pltpu.SemaphoreType.DMA,
            pltpu.SemaphoreType.DMA,
        ],
    )
    def body(idx_hbm, oidx_hbm, table_hbm, out_hbm,
             idx_v, oidx_v, rows_v, gsem0, gsem1, ssem0, ssem1):
        wid = lax.axis_index("s") * 2 + lax.axis_index("c")
        gsems = (gsem0, gsem1)
        ssems = (ssem0, ssem1)

        def gather_copies(b, j):
            return (table_hbm.at[idx_v.at[b, j]],
                    rows_v.at[b, pl.ds(j * GL, GL)], gsems[b])

        def scatter_copies(b, j):
            return (rows_v.at[b, pl.ds(j * GL, GL)],
                    out_hbm.at[oidx_v.at[b, j]], ssems[b])

        def fire_gather(ch, b):
            pltpu.sync_copy(idx_hbm.at[wid, pl.ds(ch * K, K)], idx_v.at[b])
            pltpu.sync_copy(oidx_hbm.at[wid, pl.ds(ch * K, K)], oidx_v.at[b])
            for j in range(K):
                pltpu.async_copy(*gather_copies(b, j))

        def drain_gather(b):
            for j in range(K):
                pltpu.make_async_copy(*gather_copies(b, j)).wait()

        def fire_scatter(b):
            for j in range(K):
                pltpu.async_copy(*scatter_copies(b, j))

        def drain_scatter(b):
            for j in range(K):
                pltpu.make_async_copy(*scatter_copies(b, j)).wait()

        fire_gather(0, 0)

        @pl.loop(0, NCH // 2)
        def _step(i):
            for b in (0, 1):
                ch = 2 * i + b
                nb = 1 - b
                drain_gather(b)
                fire_scatter(b)

                @pl.when(ch + 1 < NCH)
                def _prefetch():
                    @pl.when(ch >= 1)
                    def _reclaim():
                        drain_scatter(nb)
                    fire_gather(ch + 1, nb)

        drain_scatter(0)
        drain_scatter(1)

    return body(idx3, oidx3, table)


def _tc_score(g, x, idx, weo, b2):
    PB = 640
    BB = PB // 10

    def body(g_ref, x_ref, idx_ref, w_ref, b_ref, oe_ref, oo_ref):
        gb = g_ref[...].astype(jnp.bfloat16)
        wb = w_ref[...]
        acc = jnp.zeros((PB, 2 * NH), jnp.float32)
        for c in range(NCHUNK):
            acc += jnp.dot(gb[c], wb[c], preferred_element_type=jnp.float32)
        h = jnp.tanh(acc + b_ref[...])
        xb3 = jnp.broadcast_to(x_ref[...][:, None, :], (BB, 10, NH))
        he3 = h[:, :NH].reshape(BB, 10, NH)
        ho3 = h[:, NH:].reshape(BB, 10, NH)
        de = jnp.sum(he3 * xb3, axis=-1)
        do = jnp.sum(ho3 * xb3, axis=-1)
        az = jnp.all(idx_ref[...] == 0, axis=-1).astype(jnp.float32)
        dsel = lax.broadcasted_iota(jnp.int32, (N_DIST, 10), 0)
        ksel = lax.broadcasted_iota(jnp.int32, (N_DIST, 10), 1)
        se = (dsel == 2 * ksel).astype(jnp.float32)
        so = (dsel == 2 * ksel + 1).astype(jnp.float32)
        me = jnp.dot(az, se, preferred_element_type=jnp.float32) > 0.5
        mo = jnp.dot(az, so, preferred_element_type=jnp.float32) > 0.5
        oe_ref[...] = jnp.where(me, -jnp.inf, de)
        oo_ref[...] = jnp.where(mo, -jnp.inf, do)

    return pl.pallas_call(
        body,
        grid=(PAIRS_H // PB,),
        in_specs=[
            pl.BlockSpec((NCHUNK, PB, NH), lambda i: (0, i, 0)),
            pl.BlockSpec((BB, NH), lambda i: (i, 0)),
            pl.BlockSpec((BB, N_DIST, N_FEAT), lambda i: (i, 0, 0)),
            pl.BlockSpec((NCHUNK, NH, 2 * NH), lambda i: (0, 0, 0)),
            pl.BlockSpec((1, 2 * NH), lambda i: (0, 0)),
        ],
        out_specs=[
            pl.BlockSpec((BB, 10), lambda i: (i, 0)),
            pl.BlockSpec((BB, 10), lambda i: (i, 0)),
        ],
        out_shape=[
            jax.ShapeDtypeStruct((BS_H, 10), jnp.float32),
            jax.ShapeDtypeStruct((BS_H, 10), jnp.float32),
        ],
    )(g, x, idx, weo, b2)


def kernel(x, _input, table, W, b):
    idx4 = _input.reshape(NHALF, NW, KROWS, GL)
    oidx3 = _dest_rows()

    zeros = jnp.zeros_like(W)
    weo = jnp.concatenate(
        [jnp.concatenate([W, zeros], axis=0).reshape(NCHUNK, NH, NH),
         jnp.concatenate([zeros, W], axis=0).reshape(NCHUNK, NH, NH)],
        axis=-1,
    ).astype(jnp.bfloat16)
    b2 = jnp.concatenate([b, b]).reshape(1, 2 * NH)

    outs = []
    for h in range(NHALF):
        g = _sc_gather(idx4[h], oidx3, table).reshape(NCHUNK, PAIRS_H, NH)
        oe, oo = _tc_score(
            g,
            lax.slice_in_dim(x, h * BS_H, (h + 1) * BS_H),
            lax.slice_in_dim(_input, h * BS_H, (h + 1) * BS_H),
            weo, b2)
        outs.append(jnp.stack([oe, oo], axis=-1).reshape(BS_H, N_DIST))
    return jnp.concatenate(outs, axis=0)
